# baseline (speedup 1.0000x reference)
import jax
import jax.numpy as jnp
from jax.experimental import pallas as pl
from jax.experimental.pallas import tpu as pltpu

SIZE = 65536
DIM = 128
BATCH = 4096
GROUPS = 16
GROWS = DIM // GROUPS
NRING = 8


def _enqueue_body(out_hbm, bank_hbm, out_copy_hbm, bank_copy_hbm, nb_hbm,
                  xb, xt, bufs, sem_x, sem_oc, sem_hd, sem_in, sem_bc, sem_tl):
    stage_x = pltpu.make_async_copy(out_hbm, xb, sem_x)
    stage_x.start()

    def _rows(ref, g):
        return ref.at[pl.ds(g * GROWS, GROWS), :]

    def _tail(ref, g):
        return ref.at[pl.ds(g * GROWS, GROWS), pl.ds(BATCH, SIZE - BATCH)]

    ins = [None] * GROUPS
    for g in range(NRING):
        ins[g] = pltpu.make_async_copy(
            _rows(bank_hbm, g), bufs[g], sem_in[g])
        ins[g].start()

    stage_x.wait()
    xt[...] = xb[...].T
    oc = pltpu.make_async_copy(xb, out_copy_hbm, sem_oc)
    oc.start()
    hd = pltpu.make_async_copy(
        xt, nb_hbm.at[:, pl.ds(0, BATCH)], sem_hd)
    hd.start()

    bcs = [None] * GROUPS
    tls = [None] * GROUPS
    for g in range(GROUPS):
        b = g % NRING
        ins[g].wait()
        bcs[g] = pltpu.make_async_copy(
            bufs[b], _rows(bank_copy_hbm, g), sem_bc[b])
        bcs[g].start()
        tls[g] = pltpu.make_async_copy(
            bufs[b].at[:, pl.ds(BATCH, SIZE - BATCH)],
            _tail(nb_hbm, g), sem_tl[b])
        tls[g].start()
        nxt = g + 1
        if nxt < GROUPS and nxt >= NRING:
            bcs[nxt - NRING].wait()
            tls[nxt - NRING].wait()
            nb_slot = nxt % NRING
            ins[nxt] = pltpu.make_async_copy(
                _rows(bank_hbm, nxt), bufs[nb_slot], sem_in[nb_slot])
            ins[nxt].start()

    for g in range(GROUPS - NRING, GROUPS):
        bcs[g].wait()
        tls[g].wait()
    oc.wait()
    hd.wait()


def kernel(output, labels, update, bank, label):
    out_copy, bank_copy, new_bank = pl.pallas_call(
        _enqueue_body,
        in_specs=[
            pl.BlockSpec(memory_space=pl.ANY),
            pl.BlockSpec(memory_space=pl.ANY),
        ],
        out_specs=[
            pl.BlockSpec(memory_space=pl.ANY),
            pl.BlockSpec(memory_space=pl.ANY),
            pl.BlockSpec(memory_space=pl.ANY),
        ],
        out_shape=[
            jax.ShapeDtypeStruct((BATCH, DIM), jnp.float32),
            jax.ShapeDtypeStruct((DIM, SIZE), jnp.float32),
            jax.ShapeDtypeStruct((DIM, SIZE), jnp.float32),
        ],
        scratch_shapes=[
            pltpu.VMEM((BATCH, DIM), jnp.float32),
            pltpu.VMEM((DIM, BATCH), jnp.float32),
            [pltpu.VMEM((GROWS, SIZE), jnp.float32) for _ in range(NRING)],
            pltpu.SemaphoreType.DMA,
            pltpu.SemaphoreType.DMA,
            pltpu.SemaphoreType.DMA,
            [pltpu.SemaphoreType.DMA for _ in range(NRING)],
            [pltpu.SemaphoreType.DMA for _ in range(NRING)],
            [pltpu.SemaphoreType.DMA for _ in range(NRING)],
        ],
    )(output, bank)
    return (out_copy, bank_copy, new_bank)


# speedup vs baseline: 1.2167x; 1.2167x over previous
import jax
import jax.numpy as jnp
from jax.experimental import pallas as pl
from jax.experimental.pallas import tpu as pltpu

SIZE = 65536
DIM = 128
BATCH = 4096
GROUPS = 4
GROWS = DIM // GROUPS
NRING = 3


def _enqueue_body(out_hbm, bank_hbm, out_copy_hbm, bank_copy_hbm, nb_hbm,
                  xb, xt, bufs, sem_x, sem_oc, sem_hd, sem_in, sem_bc, sem_tl):
    stage_x = pltpu.make_async_copy(out_hbm, xb, sem_x)
    stage_x.start()

    def _rows(ref, g):
        return ref.at[pl.ds(g * GROWS, GROWS), :]

    def _tail(ref, g):
        return ref.at[pl.ds(g * GROWS, GROWS), pl.ds(BATCH, SIZE - BATCH)]

    ins = [None] * GROUPS
    for g in range(NRING):
        ins[g] = pltpu.make_async_copy(
            _rows(bank_hbm, g), bufs[g], sem_in[g])
        ins[g].start()

    stage_x.wait()
    xt[...] = xb[...].T
    oc = pltpu.make_async_copy(xb, out_copy_hbm, sem_oc)
    oc.start()
    hd = pltpu.make_async_copy(
        xt, nb_hbm.at[:, pl.ds(0, BATCH)], sem_hd)
    hd.start()

    bcs = [None] * GROUPS
    tls = [None] * GROUPS
    for g in range(GROUPS):
        b = g % NRING
        ins[g].wait()
        bcs[g] = pltpu.make_async_copy(
            bufs[b], _rows(bank_copy_hbm, g), sem_bc[b])
        bcs[g].start()
        tls[g] = pltpu.make_async_copy(
            bufs[b].at[:, pl.ds(BATCH, SIZE - BATCH)],
            _tail(nb_hbm, g), sem_tl[b])
        tls[g].start()
        nxt = g + 1
        if nxt < GROUPS and nxt >= NRING:
            bcs[nxt - NRING].wait()
            tls[nxt - NRING].wait()
            nb_slot = nxt % NRING
            ins[nxt] = pltpu.make_async_copy(
                _rows(bank_hbm, nxt), bufs[nb_slot], sem_in[nb_slot])
            ins[nxt].start()

    for g in range(GROUPS - NRING, GROUPS):
        bcs[g].wait()
        tls[g].wait()
    oc.wait()
    hd.wait()


def kernel(output, labels, update, bank, label):
    out_copy, bank_copy, new_bank = pl.pallas_call(
        _enqueue_body,
        in_specs=[
            pl.BlockSpec(memory_space=pl.ANY),
            pl.BlockSpec(memory_space=pl.ANY),
        ],
        out_specs=[
            pl.BlockSpec(memory_space=pl.ANY),
            pl.BlockSpec(memory_space=pl.ANY),
            pl.BlockSpec(memory_space=pl.ANY),
        ],
        out_shape=[
            jax.ShapeDtypeStruct((BATCH, DIM), jnp.float32),
            jax.ShapeDtypeStruct((DIM, SIZE), jnp.float32),
            jax.ShapeDtypeStruct((DIM, SIZE), jnp.float32),
        ],
        scratch_shapes=[
            pltpu.VMEM((BATCH, DIM), jnp.float32),
            pltpu.VMEM((DIM, BATCH), jnp.float32),
            [pltpu.VMEM((GROWS, SIZE), jnp.float32) for _ in range(NRING)],
            pltpu.SemaphoreType.DMA,
            pltpu.SemaphoreType.DMA,
            pltpu.SemaphoreType.DMA,
            [pltpu.SemaphoreType.DMA for _ in range(NRING)],
            [pltpu.SemaphoreType.DMA for _ in range(NRING)],
            [pltpu.SemaphoreType.DMA for _ in range(NRING)],
        ],
    )(output, bank)
    return (out_copy, bank_copy, new_bank)


# speedup vs baseline: 1.2342x; 1.0144x over previous
import jax
import jax.numpy as jnp
from jax.experimental import pallas as pl
from jax.experimental.pallas import tpu as pltpu

SIZE = 65536
DIM = 128
BATCH = 4096
GROUPS = 2
GROWS = DIM // GROUPS
NRING = 2


def _enqueue_body(out_hbm, bank_hbm, out_copy_hbm, bank_copy_hbm, nb_hbm,
                  xb, xt, bufs, sem_x, sem_oc, sem_hd, sem_in, sem_bc, sem_tl):
    stage_x = pltpu.make_async_copy(out_hbm, xb, sem_x)
    stage_x.start()

    def _rows(ref, g):
        return ref.at[pl.ds(g * GROWS, GROWS), :]

    def _tail(ref, g):
        return ref.at[pl.ds(g * GROWS, GROWS), pl.ds(BATCH, SIZE - BATCH)]

    ins = [None] * GROUPS
    for g in range(NRING):
        ins[g] = pltpu.make_async_copy(
            _rows(bank_hbm, g), bufs[g], sem_in[g])
        ins[g].start()

    stage_x.wait()
    xt[...] = xb[...].T
    oc = pltpu.make_async_copy(xb, out_copy_hbm, sem_oc)
    oc.start()
    hd = pltpu.make_async_copy(
        xt, nb_hbm.at[:, pl.ds(0, BATCH)], sem_hd)
    hd.start()

    bcs = [None] * GROUPS
    tls = [None] * GROUPS
    for g in range(GROUPS):
        b = g % NRING
        ins[g].wait()
        bcs[g] = pltpu.make_async_copy(
            bufs[b], _rows(bank_copy_hbm, g), sem_bc[b])
        bcs[g].start()
        tls[g] = pltpu.make_async_copy(
            bufs[b].at[:, pl.ds(BATCH, SIZE - BATCH)],
            _tail(nb_hbm, g), sem_tl[b])
        tls[g].start()
        nxt = g + 1
        if nxt < GROUPS and nxt >= NRING:
            bcs[nxt - NRING].wait()
            tls[nxt - NRING].wait()
            nb_slot = nxt % NRING
            ins[nxt] = pltpu.make_async_copy(
                _rows(bank_hbm, nxt), bufs[nb_slot], sem_in[nb_slot])
            ins[nxt].start()

    for g in range(GROUPS - NRING, GROUPS):
        bcs[g].wait()
        tls[g].wait()
    oc.wait()
    hd.wait()


def kernel(output, labels, update, bank, label):
    out_copy, bank_copy, new_bank = pl.pallas_call(
        _enqueue_body,
        in_specs=[
            pl.BlockSpec(memory_space=pl.ANY),
            pl.BlockSpec(memory_space=pl.ANY),
        ],
        out_specs=[
            pl.BlockSpec(memory_space=pl.ANY),
            pl.BlockSpec(memory_space=pl.ANY),
            pl.BlockSpec(memory_space=pl.ANY),
        ],
        out_shape=[
            jax.ShapeDtypeStruct((BATCH, DIM), jnp.float32),
            jax.ShapeDtypeStruct((DIM, SIZE), jnp.float32),
            jax.ShapeDtypeStruct((DIM, SIZE), jnp.float32),
        ],
        scratch_shapes=[
            pltpu.VMEM((BATCH, DIM), jnp.float32),
            pltpu.VMEM((DIM, BATCH), jnp.float32),
            [pltpu.VMEM((GROWS, SIZE), jnp.float32) for _ in range(NRING)],
            pltpu.SemaphoreType.DMA,
            pltpu.SemaphoreType.DMA,
            pltpu.SemaphoreType.DMA,
            [pltpu.SemaphoreType.DMA for _ in range(NRING)],
            [pltpu.SemaphoreType.DMA for _ in range(NRING)],
            [pltpu.SemaphoreType.DMA for _ in range(NRING)],
        ],
    )(output, bank)
    return (out_copy, bank_copy, new_bank)
